# baseline (speedup 1.0000x reference)
import functools

import jax
import jax.numpy as jnp
from jax import lax
from jax.experimental import pallas as pl
from jax.experimental.pallas import tpu as pltpu
from jax.experimental.pallas import tpu_sc as plsc

_LANES = 16
_NW = 32


def _lg(ref, idxs, mask):
    if mask is None:
        return plsc.load_gather(ref, idxs)
    return plsc.load_gather(ref, idxs, mask=mask)


def _st(ref, idxs, x, mask):
    if mask is None:
        plsc.store_scatter(ref, idxs, x)
    else:
        plsc.store_scatter(ref, idxs, x, mask=mask)


def _ln_embed_sc(word_i, posi_i, ag_i, word_table, posi_table, ag_table,
                 ln_gamma, ln_beta):
    b, l = word_i.shape
    hid = word_table.shape[1]
    n_posi = posi_table.shape[0]
    n_ag = ag_table.shape[0]
    rows_w = b // _NW
    n_grp = -(-l // _LANES)
    n_full = l // _LANES
    l_pad = n_grp * _LANES

    mesh = plsc.VectorSubcoreMesh(core_axis_name="c", subcore_axis_name="s")

    @functools.partial(
        pl.kernel,
        mesh=mesh,
        out_type=jax.ShapeDtypeStruct((b, l, hid), jnp.float32),
        scratch_types=[
            [pltpu.VMEM((3, l_pad), jnp.int32) for _ in range(2)],
            [pltpu.VMEM((l, hid), jnp.float32) for _ in range(2)],
            [pltpu.VMEM((l, hid), jnp.float32) for _ in range(2)],
            pltpu.VMEM((n_grp * hid * _LANES,), jnp.float32),
            pltpu.VMEM((n_posi, hid), jnp.float32),
            pltpu.VMEM((n_ag, hid), jnp.float32),
            pltpu.VMEM((hid,), jnp.float32),
            pltpu.VMEM((hid,), jnp.float32),
            [pltpu.SemaphoreType.DMA for _ in range(2)],
            [pltpu.SemaphoreType.DMA for _ in range(2)],
            [pltpu.SemaphoreType.DMA for _ in range(2)],
        ],
        compiler_params=pltpu.CompilerParams(use_tc_tiling_on_sc=False,
                                             needs_layout_passes=False),
    )
    def k(wi_hbm, pi_hbm, ai_hbm, wt_hbm, pt_hbm, agt_hbm, gam_hbm, bet_hbm,
          out_hbm, iv, wb, ob, xs, pt_v, agt_v, gam_v, bet_v,
          isem, gsem, osem):
        cid = lax.axis_index("c")
        sid = lax.axis_index("s")
        wid = sid * 2 + cid
        row0 = wid * rows_w

        pltpu.sync_copy(pt_hbm, pt_v)
        pltpu.sync_copy(agt_hbm, agt_v)
        pltpu.sync_copy(gam_hbm, gam_v)
        pltpu.sync_copy(bet_hbm, bet_v)

        lane = lax.iota(jnp.int32, _LANES)
        mtail = lane < (l - n_full * _LANES)

        def idx_copy(c, u):
            row = row0 + c
            return [
                pltpu.make_async_copy(wi_hbm.at[row],
                                      iv[u].at[0, pl.ds(0, l)], isem[u]),
                pltpu.make_async_copy(pi_hbm.at[row],
                                      iv[u].at[1, pl.ds(0, l)], isem[u]),
                pltpu.make_async_copy(ai_hbm.at[row],
                                      iv[u].at[2, pl.ds(0, l)], isem[u]),
            ]

        def gather(c, u):
            w = u % 2
            return [
                pltpu.make_async_copy(
                    wt_hbm.at[iv[u].at[0, pl.ds(0, 128)]],
                    wb[w].at[pl.ds(0, 128)], gsem[w]),
                pltpu.make_async_copy(
                    wt_hbm.at[iv[u].at[0, pl.ds(128, l - 128)]],
                    wb[w].at[pl.ds(128, l - 128)], gsem[w]),
            ]

        def scatter(c, u):
            return pltpu.make_async_copy(
                ob[u % 2], out_hbm.at[row0 + c], osem[u % 2])

        def compute(c, u):
            wbf = wb[u % 2]
            obf = ob[u % 2]
            scale = []
            shift = []
            tokl = []
            gxl = []
            for g in range(n_grp):
                mk = None if g < n_full else mtail
                tl = lane + g * _LANES
                tokl.append(tl)
                gxl.append(lane + g * (hid * _LANES))
                pidx = iv[u][1, pl.ds(g * _LANES, _LANES)]
                agidx = iv[u][2, pl.ds(g * _LANES, _LANES)]
                zeros = jnp.zeros((_LANES,), jnp.float32)

                def jstep(jj, acc, g=g, tl=tl, pidx=pidx, agidx=agidx,
                          mk=mk):
                    s0, s1 = acc
                    e = lax.bitwise_and(lane + jj, hid - 1)
                    x = (_lg(wbf, [tl, e], mk)
                         + _lg(pt_v, [pidx, e], mk)
                         + _lg(agt_v, [agidx, e], mk))
                    _st(xs, [gxl[g] + lax.shift_left(e, 4)], x, mk)
                    return (s0 + x, s1 + x * x)

                s0, s1 = lax.fori_loop(0, hid, jstep, (zeros, zeros),
                                       unroll=2)
                mean = s0 * (1.0 / hid)
                var = s1 * (1.0 / hid) - mean * mean + 1e-12
                bits = lax.bitcast_convert_type(var, jnp.int32)
                y = lax.bitcast_convert_type(
                    jnp.int32(0x5F3759DF) - lax.shift_right_arithmetic(bits, 1),
                    jnp.float32)
                for _ in range(3):
                    y = y * (1.5 - 0.5 * var * y * y)
                scale.append(y)
                shift.append(-mean * y)

            def jnorm(jj, carry2):
                e = lax.bitwise_and(lane + jj, hid - 1)
                gam = plsc.load_gather(gam_v, [e])
                bet = plsc.load_gather(bet_v, [e])
                e16 = lax.shift_left(e, 4)
                for g in range(n_grp):
                    mk = None if g < n_full else mtail
                    x = _lg(xs, [gxl[g] + e16], mk)
                    o = (x * scale[g] + shift[g]) * gam + bet
                    _st(obf, [tokl[g], e], o, mk)
                return carry2

            lax.fori_loop(0, hid, jnorm, 0, unroll=2)

        n_chunks = rows_w

        for u in range(2):
            for d in idx_copy(u, u):
                d.start()
        for d in idx_copy(0, 0):
            d.wait()
        for d in gather(0, 0):
            d.start()

        def pair_body(t, carry):
            for u in range(2):
                c = t * 2 + u
                for d in gather(c, u):
                    d.wait()

                @pl.when(c + 1 < n_chunks)
                def _(u=u, c=c):
                    u1 = (u + 1) % 2
                    for d in idx_copy(c + 1, u1):
                        d.wait()
                    for d in gather(c + 1, u1):
                        d.start()

                @pl.when(c >= 2)
                def _(u=u, c=c):
                    scatter(c - 2, u).wait()

                compute(c, u)
                scatter(c, u).start()

                @pl.when(c + 2 < n_chunks)
                def _(u=u, c=c):
                    for d in idx_copy(c + 2, u):
                        d.start()
            return carry

        lax.fori_loop(0, n_chunks // 2, pair_body, 0)
        scatter(n_chunks - 2, 0).wait()
        scatter(n_chunks - 1, 1).wait()

    return k(word_i, posi_i, ag_i, word_table, posi_table, ag_table,
             ln_gamma, ln_beta)


def kernel(word_ids, posi_ids, age_ids, gender_ids, word_table, posi_table,
           age_table, gender_table, ln_gamma, ln_beta):
    hid = word_table.shape[1]
    n_gen = gender_table.shape[0]
    ag_table = (age_table[:, None, :] + gender_table[None, :, :]
                ).reshape(-1, hid)
    return _ln_embed_sc(
        word_ids.astype(jnp.int32),
        posi_ids.astype(jnp.int32),
        (age_ids * n_gen + gender_ids).astype(jnp.int32),
        word_table, posi_table, ag_table, ln_gamma, ln_beta)


# speedup vs baseline: 1.0010x; 1.0010x over previous
import functools

import jax
import jax.numpy as jnp
from jax import lax
from jax.experimental import pallas as pl
from jax.experimental.pallas import tpu as pltpu
from jax.experimental.pallas import tpu_sc as plsc

_LANES = 16
_NW = 32


def _lg(ref, idxs, mask):
    if mask is None:
        return plsc.load_gather(ref, idxs)
    return plsc.load_gather(ref, idxs, mask=mask)


def _st(ref, idxs, x, mask):
    if mask is None:
        plsc.store_scatter(ref, idxs, x)
    else:
        plsc.store_scatter(ref, idxs, x, mask=mask)


def _ln_embed_sc(word_i, posi_i, ag_i, word_table, posi_table, ag_table,
                 ln_gamma, ln_beta):
    b, l = word_i.shape
    hid = word_table.shape[1]
    n_posi = posi_table.shape[0]
    n_ag = ag_table.shape[0]
    rows_w = b // _NW
    n_grp = -(-l // _LANES)
    n_full = l // _LANES
    l_pad = n_grp * _LANES

    mesh = plsc.VectorSubcoreMesh(core_axis_name="c", subcore_axis_name="s")

    @functools.partial(
        pl.kernel,
        mesh=mesh,
        out_type=jax.ShapeDtypeStruct((b, l, hid), jnp.float32),
        scratch_types=[
            [pltpu.VMEM((3, l_pad), jnp.int32) for _ in range(2)],
            [pltpu.VMEM((l, hid), jnp.float32) for _ in range(2)],
            [pltpu.VMEM((l, hid), jnp.float32) for _ in range(2)],
            pltpu.VMEM((n_grp * hid * _LANES,), jnp.float32),
            pltpu.VMEM((n_posi, hid), jnp.float32),
            pltpu.VMEM((n_ag, hid), jnp.float32),
            pltpu.VMEM((hid,), jnp.float32),
            pltpu.VMEM((hid,), jnp.float32),
            [pltpu.SemaphoreType.DMA for _ in range(2)],
            [pltpu.SemaphoreType.DMA for _ in range(2)],
            [pltpu.SemaphoreType.DMA for _ in range(2)],
        ],
        compiler_params=pltpu.CompilerParams(use_tc_tiling_on_sc=False,
                                             needs_layout_passes=False),
    )
    def k(wi_hbm, pi_hbm, ai_hbm, wt_hbm, pt_hbm, agt_hbm, gam_hbm, bet_hbm,
          out_hbm, iv, wb, ob, xs, pt_v, agt_v, gam_v, bet_v,
          isem, gsem, osem):
        cid = lax.axis_index("c")
        sid = lax.axis_index("s")
        wid = sid * 2 + cid
        row0 = wid * rows_w

        pltpu.sync_copy(pt_hbm, pt_v)
        pltpu.sync_copy(agt_hbm, agt_v)
        pltpu.sync_copy(gam_hbm, gam_v)
        pltpu.sync_copy(bet_hbm, bet_v)

        lane = lax.iota(jnp.int32, _LANES)
        mtail = lane < (l - n_full * _LANES)

        def idx_copy(c, u):
            row = row0 + c
            return [
                pltpu.make_async_copy(wi_hbm.at[row],
                                      iv[u].at[0, pl.ds(0, l)], isem[u]),
                pltpu.make_async_copy(pi_hbm.at[row],
                                      iv[u].at[1, pl.ds(0, l)], isem[u]),
                pltpu.make_async_copy(ai_hbm.at[row],
                                      iv[u].at[2, pl.ds(0, l)], isem[u]),
            ]

        def gather(c, u):
            w = u % 2
            return [
                pltpu.make_async_copy(
                    wt_hbm.at[iv[u].at[0, pl.ds(0, 128)]],
                    wb[w].at[pl.ds(0, 128)], gsem[w]),
                pltpu.make_async_copy(
                    wt_hbm.at[iv[u].at[0, pl.ds(128, l - 128)]],
                    wb[w].at[pl.ds(128, l - 128)], gsem[w]),
            ]

        def scatter(c, u):
            return pltpu.make_async_copy(
                ob[u % 2], out_hbm.at[row0 + c], osem[u % 2])

        def compute(c, u):
            wbf = wb[u % 2]
            obf = ob[u % 2]
            scale = []
            shift = []
            tokl = []
            gxl = []
            for g in range(n_grp):
                mk = None if g < n_full else mtail
                tl = lane + g * _LANES
                tokl.append(tl)
                gxl.append(lane + g * (hid * _LANES))
                pidx = iv[u][1, pl.ds(g * _LANES, _LANES)]
                agidx = iv[u][2, pl.ds(g * _LANES, _LANES)]
                zeros = jnp.zeros((_LANES,), jnp.float32)

                def jstep(jj, acc, g=g, tl=tl, pidx=pidx, agidx=agidx,
                          mk=mk):
                    s0, s1 = acc
                    e = lax.bitwise_and(lane + jj, hid - 1)
                    x = (_lg(wbf, [tl, e], mk)
                         + _lg(pt_v, [pidx, e], mk)
                         + _lg(agt_v, [agidx, e], mk))
                    _st(xs, [gxl[g] + lax.shift_left(e, 4)], x, mk)
                    return (s0 + x, s1 + x * x)

                s0, s1 = lax.fori_loop(0, hid, jstep, (zeros, zeros),
                                       unroll=4)
                mean = s0 * (1.0 / hid)
                var = s1 * (1.0 / hid) - mean * mean + 1e-12
                bits = lax.bitcast_convert_type(var, jnp.int32)
                y = lax.bitcast_convert_type(
                    jnp.int32(0x5F3759DF) - lax.shift_right_arithmetic(bits, 1),
                    jnp.float32)
                for _ in range(3):
                    y = y * (1.5 - 0.5 * var * y * y)
                scale.append(y)
                shift.append(-mean * y)

            def jnorm(jj, carry2):
                e = lax.bitwise_and(lane + jj, hid - 1)
                gam = plsc.load_gather(gam_v, [e])
                bet = plsc.load_gather(bet_v, [e])
                e16 = lax.shift_left(e, 4)
                for g in range(n_grp):
                    mk = None if g < n_full else mtail
                    x = _lg(xs, [gxl[g] + e16], mk)
                    o = (x * scale[g] + shift[g]) * gam + bet
                    _st(obf, [tokl[g], e], o, mk)
                return carry2

            lax.fori_loop(0, hid, jnorm, 0, unroll=4)

        n_chunks = rows_w

        for u in range(2):
            for d in idx_copy(u, u):
                d.start()
        for d in idx_copy(0, 0):
            d.wait()
        for d in gather(0, 0):
            d.start()

        def pair_body(t, carry):
            for u in range(2):
                c = t * 2 + u
                for d in gather(c, u):
                    d.wait()

                @pl.when(c + 1 < n_chunks)
                def _(u=u, c=c):
                    u1 = (u + 1) % 2
                    for d in idx_copy(c + 1, u1):
                        d.wait()
                    for d in gather(c + 1, u1):
                        d.start()

                @pl.when(c >= 2)
                def _(u=u, c=c):
                    scatter(c - 2, u).wait()

                compute(c, u)
                scatter(c, u).start()

                @pl.when(c + 2 < n_chunks)
                def _(u=u, c=c):
                    for d in idx_copy(c + 2, u):
                        d.start()
            return carry

        lax.fori_loop(0, n_chunks // 2, pair_body, 0)
        scatter(n_chunks - 2, 0).wait()
        scatter(n_chunks - 1, 1).wait()

    return k(word_i, posi_i, ag_i, word_table, posi_table, ag_table,
             ln_gamma, ln_beta)


def kernel(word_ids, posi_ids, age_ids, gender_ids, word_table, posi_table,
           age_table, gender_table, ln_gamma, ln_beta):
    hid = word_table.shape[1]
    n_gen = gender_table.shape[0]
    ag_table = (age_table[:, None, :] + gender_table[None, :, :]
                ).reshape(-1, hid)
    return _ln_embed_sc(
        word_ids.astype(jnp.int32),
        posi_ids.astype(jnp.int32),
        (age_ids * n_gen + gender_ids).astype(jnp.int32),
        word_table, posi_table, ag_table, ln_gamma, ln_beta)


# speedup vs baseline: 1.0049x; 1.0039x over previous
import functools

import jax
import jax.numpy as jnp
from jax import lax
from jax.experimental import pallas as pl
from jax.experimental.pallas import tpu as pltpu
from jax.experimental.pallas import tpu_sc as plsc

_LANES = 16
_NW = 32


def _lg(ref, idxs, mask):
    if mask is None:
        return plsc.load_gather(ref, idxs)
    return plsc.load_gather(ref, idxs, mask=mask)


def _st(ref, idxs, x, mask):
    if mask is None:
        plsc.store_scatter(ref, idxs, x)
    else:
        plsc.store_scatter(ref, idxs, x, mask=mask)


def _ln_embed_sc(word_i, posi_i, ag_i, word_table, posi_table, ag_table,
                 ln_gamma, ln_beta):
    b, l = word_i.shape
    hid = word_table.shape[1]
    n_posi = posi_table.shape[0]
    n_ag = ag_table.shape[0]
    rows_w = b // _NW
    n_grp = -(-l // _LANES)
    n_full = l // _LANES
    l_pad = n_grp * _LANES

    mesh = plsc.VectorSubcoreMesh(core_axis_name="c", subcore_axis_name="s")

    @functools.partial(
        pl.kernel,
        mesh=mesh,
        out_type=jax.ShapeDtypeStruct((b, l, hid), jnp.float32),
        scratch_types=[
            [pltpu.VMEM((3, l_pad), jnp.int32) for _ in range(2)],
            [pltpu.VMEM((l, hid), jnp.float32) for _ in range(2)],
            [pltpu.VMEM((l, hid), jnp.float32) for _ in range(2)],
            pltpu.VMEM((n_grp, hid, _LANES), jnp.float32),
            pltpu.VMEM((n_posi, hid), jnp.float32),
            pltpu.VMEM((n_ag, hid), jnp.float32),
            pltpu.VMEM((hid,), jnp.float32),
            pltpu.VMEM((hid,), jnp.float32),
            [pltpu.SemaphoreType.DMA for _ in range(2)],
            [pltpu.SemaphoreType.DMA for _ in range(2)],
            [pltpu.SemaphoreType.DMA for _ in range(2)],
        ],
        compiler_params=pltpu.CompilerParams(use_tc_tiling_on_sc=False,
                                             needs_layout_passes=False),
    )
    def k(wi_hbm, pi_hbm, ai_hbm, wt_hbm, pt_hbm, agt_hbm, gam_hbm, bet_hbm,
          out_hbm, iv, wb, ob, xs, pt_v, agt_v, gam_v, bet_v,
          isem, gsem, osem):
        cid = lax.axis_index("c")
        sid = lax.axis_index("s")
        wid = sid * 2 + cid
        row0 = wid * rows_w

        pltpu.sync_copy(pt_hbm, pt_v)
        pltpu.sync_copy(agt_hbm, agt_v)
        pltpu.sync_copy(gam_hbm, gam_v)
        pltpu.sync_copy(bet_hbm, bet_v)

        lane = lax.iota(jnp.int32, _LANES)
        mtail = lane < (l - n_full * _LANES)

        def idx_copy(c, u):
            row = row0 + c
            return [
                pltpu.make_async_copy(wi_hbm.at[row],
                                      iv[u].at[0, pl.ds(0, l)], isem[u]),
                pltpu.make_async_copy(pi_hbm.at[row],
                                      iv[u].at[1, pl.ds(0, l)], isem[u]),
                pltpu.make_async_copy(ai_hbm.at[row],
                                      iv[u].at[2, pl.ds(0, l)], isem[u]),
            ]

        def gather(c, u):
            w = u % 2
            return [
                pltpu.make_async_copy(
                    wt_hbm.at[iv[u].at[0, pl.ds(0, 128)]],
                    wb[w].at[pl.ds(0, 128)], gsem[w]),
                pltpu.make_async_copy(
                    wt_hbm.at[iv[u].at[0, pl.ds(128, l - 128)]],
                    wb[w].at[pl.ds(128, l - 128)], gsem[w]),
            ]

        def scatter(c, u):
            return pltpu.make_async_copy(
                ob[u % 2], out_hbm.at[row0 + c], osem[u % 2])

        def compute(c, u):
            wbf = wb[u % 2]
            obf = ob[u % 2]
            scale = []
            shift = []
            tokl = []
            gxl = []
            for g in range(n_grp):
                mk = None if g < n_full else mtail
                tl = lane + g * _LANES
                tokl.append(tl)
                gxl.append(lane + g * (hid * _LANES))
                pidx = iv[u][1, pl.ds(g * _LANES, _LANES)]
                agidx = iv[u][2, pl.ds(g * _LANES, _LANES)]
                zeros = jnp.zeros((_LANES,), jnp.float32)

                def jstep(jj, acc, g=g, tl=tl, pidx=pidx, agidx=agidx,
                          mk=mk):
                    s0, s1 = acc
                    e = lax.bitwise_and(lane + jj, hid - 1)
                    x = (_lg(wbf, [tl, e], mk)
                         + _lg(pt_v, [pidx, e], mk)
                         + _lg(agt_v, [agidx, e], mk))
                    xs[g, jj, :] = x
                    return (s0 + x, s1 + x * x)

                s0, s1 = lax.fori_loop(0, hid, jstep, (zeros, zeros),
                                       unroll=4)
                mean = s0 * (1.0 / hid)
                var = s1 * (1.0 / hid) - mean * mean + 1e-12
                bits = lax.bitcast_convert_type(var, jnp.int32)
                y = lax.bitcast_convert_type(
                    jnp.int32(0x5F3759DF) - lax.shift_right_arithmetic(bits, 1),
                    jnp.float32)
                for _ in range(3):
                    y = y * (1.5 - 0.5 * var * y * y)
                scale.append(y)
                shift.append(-mean * y)

            def jnorm(jj, carry2):
                e = lax.bitwise_and(lane + jj, hid - 1)
                gam = plsc.load_gather(gam_v, [e])
                bet = plsc.load_gather(bet_v, [e])
                for g in range(n_grp):
                    mk = None if g < n_full else mtail
                    x = xs[g, jj, :]
                    o = (x * scale[g] + shift[g]) * gam + bet
                    _st(obf, [tokl[g], e], o, mk)
                return carry2

            lax.fori_loop(0, hid, jnorm, 0, unroll=4)

        n_chunks = rows_w

        for u in range(2):
            for d in idx_copy(u, u):
                d.start()
        for d in idx_copy(0, 0):
            d.wait()
        for d in gather(0, 0):
            d.start()

        def pair_body(t, carry):
            for u in range(2):
                c = t * 2 + u
                for d in gather(c, u):
                    d.wait()

                @pl.when(c + 1 < n_chunks)
                def _(u=u, c=c):
                    u1 = (u + 1) % 2
                    for d in idx_copy(c + 1, u1):
                        d.wait()
                    for d in gather(c + 1, u1):
                        d.start()

                @pl.when(c >= 2)
                def _(u=u, c=c):
                    scatter(c - 2, u).wait()

                compute(c, u)
                scatter(c, u).start()

                @pl.when(c + 2 < n_chunks)
                def _(u=u, c=c):
                    for d in idx_copy(c + 2, u):
                        d.start()
            return carry

        lax.fori_loop(0, n_chunks // 2, pair_body, 0)
        scatter(n_chunks - 2, 0).wait()
        scatter(n_chunks - 1, 1).wait()

    return k(word_i, posi_i, ag_i, word_table, posi_table, ag_table,
             ln_gamma, ln_beta)


def kernel(word_ids, posi_ids, age_ids, gender_ids, word_table, posi_table,
           age_table, gender_table, ln_gamma, ln_beta):
    hid = word_table.shape[1]
    n_gen = gender_table.shape[0]
    ag_table = (age_table[:, None, :] + gender_table[None, :, :]
                ).reshape(-1, hid)
    return _ln_embed_sc(
        word_ids.astype(jnp.int32),
        posi_ids.astype(jnp.int32),
        (age_ids * n_gen + gender_ids).astype(jnp.int32),
        word_table, posi_table, ag_table, ln_gamma, ln_beta)
